# baseline (speedup 1.0000x reference)
import functools
import jax
import jax.numpy as jnp
from jax import lax
from jax.experimental import pallas as pl
from jax.experimental.pallas import tpu as pltpu
from jax.experimental.pallas import tpu_sc as plsc

_B, _T, _D = 4, 2048, 1024
_E, _K, _DFF = 8, 2, 4096
_N = _B * _T
_A = _N * _K
_TM = 512
_P = _A + _E * _TM
_NT = _P // _TM
_F = 1024
_NF = _DFF // _F
_EP = 128


def _router_body(x_ref, w_ref, b_ref, ti_ref, tv_ref, imp_ref):
    i = pl.program_id(0)
    x = x_ref[...]
    logits = jnp.dot(x, w_ref[...], preferred_element_type=jnp.float32)
    logits = logits + b_ref[...]
    col = lax.broadcasted_iota(jnp.int32, logits.shape, 1)
    logits = jnp.where(col < _E, logits, -1e30)
    m = jnp.max(logits, axis=-1, keepdims=True)
    p = jnp.exp(logits - m)
    probs = p / jnp.sum(p, axis=-1, keepdims=True)

    v0 = jnp.max(probs, axis=-1, keepdims=True)
    e0 = jnp.min(jnp.where(probs == v0, col, _EP), axis=-1, keepdims=True)
    probs2 = jnp.where(col == e0, -1.0, probs)
    v1 = jnp.max(probs2, axis=-1, keepdims=True)
    e1 = jnp.min(jnp.where(probs2 == v1, col, _EP), axis=-1, keepdims=True)
    s = v0 + v1
    w0 = v0 / s
    w1 = v1 / s

    ti_ref[...] = jnp.where(col == 0, e0, jnp.where(col == 1, e1, 0))
    tv_ref[...] = jnp.where(col == 0, w0, jnp.where(col == 1, w1, 0.0))

    @pl.when(i == 0)
    def _():
        imp_ref[...] = jnp.zeros_like(imp_ref)
    imp_ref[...] += jnp.sum(probs, axis=0, keepdims=True)


def _run_router(x_flat, router_Wp, router_bp):
    tb = 1024
    return pl.pallas_call(
        _router_body,
        grid=(_N // tb,),
        in_specs=[
            pl.BlockSpec((tb, _D), lambda i: (i, 0)),
            pl.BlockSpec((_D, _EP), lambda i: (0, 0)),
            pl.BlockSpec((1, _EP), lambda i: (0, 0)),
        ],
        out_specs=[
            pl.BlockSpec((tb, _EP), lambda i: (i, 0)),
            pl.BlockSpec((tb, _EP), lambda i: (i, 0)),
            pl.BlockSpec((1, _EP), lambda i: (0, 0)),
        ],
        out_shape=[
            jax.ShapeDtypeStruct((_N, _EP), jnp.int32),
            jax.ShapeDtypeStruct((_N, _EP), jnp.float32),
            jax.ShapeDtypeStruct((1, _EP), jnp.float32),
        ],
    )(x_flat, router_Wp, router_bp)


def _make_sc_gather(n_rows, d, chunk, nbuf=3):
    info = plsc.get_sparse_core_info()
    nw = info.num_cores * info.num_subcores
    per_w = n_rows // nw
    n_chunks = per_w // chunk
    mesh = plsc.VectorSubcoreMesh(core_axis_name="c", subcore_axis_name="s")

    @functools.partial(
        pl.kernel, mesh=mesh,
        out_type=jax.ShapeDtypeStruct((n_rows, d), jnp.float32),
        scratch_types=[
            pltpu.VMEM((per_w,), jnp.int32),
        ] + [pltpu.VMEM((chunk, d), jnp.float32) for _ in range(nbuf)]
        + [pltpu.SemaphoreType.DMA for _ in range(2 * nbuf)],
    )
    def k(table_hbm, idx_hbm, out_hbm, idx_v, *bufs_sems):
        bufs = bufs_sems[:nbuf]
        gsems = bufs_sems[nbuf:2 * nbuf]
        ssems = bufs_sems[2 * nbuf:]
        wid = lax.axis_index("s") * info.num_cores + lax.axis_index("c")
        base = wid * per_w
        pltpu.sync_copy(idx_hbm.at[pl.ds(base, per_w)], idx_v)

        def start(c):
            b = c % nbuf
            return pltpu.async_copy(
                table_hbm.at[idx_v.at[pl.ds(c * chunk, chunk)]],
                bufs[b], gsems[b])

        copies = {}
        for c in range(min(nbuf, n_chunks)):
            copies[c] = start(c)
        for c in range(n_chunks):
            b = c % nbuf
            copies.pop(c).wait()
            st = pltpu.async_copy(
                bufs[b], out_hbm.at[pl.ds(base + c * chunk, chunk)], ssems[b])
            if c + nbuf < n_chunks:
                st.wait()
                copies[c + nbuf] = start(c + nbuf)
            else:
                st.wait()

    return k


def _ffn_body(eid_ref, x_ref, w1_ref, b1_ref, w2_ref, b2_ref,
              out_ref, acc_ref):
    f = pl.program_id(1)
    xb = x_ref[...].astype(jnp.bfloat16)
    h = jnp.dot(xb, w1_ref[0], preferred_element_type=jnp.float32)
    h = jnp.maximum(h + b1_ref[0], 0.0)
    contrib = jnp.dot(h.astype(jnp.bfloat16), w2_ref[0],
                      preferred_element_type=jnp.float32)

    @pl.when(f == 0)
    def _():
        acc_ref[...] = jnp.zeros_like(acc_ref)
    acc_ref[...] += contrib

    @pl.when(f == _NF - 1)
    def _():
        out_ref[...] = acc_ref[...] + b2_ref[0]


def _run_ffn(expert_tile, x_sorted, W1, b1, W2, b2):
    grid_spec = pltpu.PrefetchScalarGridSpec(
        num_scalar_prefetch=1,
        grid=(_NT, _NF),
        in_specs=[
            pl.BlockSpec((_TM, _D), lambda i, f, e: (i, 0)),
            pl.BlockSpec((1, _D, _F), lambda i, f, e: (e[i], 0, f)),
            pl.BlockSpec((1, 1, _F), lambda i, f, e: (e[i] * _NF + f, 0, 0)),
            pl.BlockSpec((1, _F, _D), lambda i, f, e: (e[i], f, 0)),
            pl.BlockSpec((1, 1, _D), lambda i, f, e: (e[i], 0, 0)),
        ],
        out_specs=pl.BlockSpec((_TM, _D), lambda i, f, e: (i, 0)),
        scratch_shapes=[pltpu.VMEM((_TM, _D), jnp.float32)],
    )
    return pl.pallas_call(
        _ffn_body,
        grid_spec=grid_spec,
        out_shape=jax.ShapeDtypeStruct((_P, _D), jnp.float32),
        compiler_params=pltpu.CompilerParams(
            dimension_semantics=("arbitrary", "arbitrary"),
        ),
    )(expert_tile, x_sorted, W1, b1, W2, b2)


def _make_sc_combine(n_tok, d, chunk, nbuf=2):
    info = plsc.get_sparse_core_info()
    nw = info.num_cores * info.num_subcores
    per_w = n_tok // nw
    n_chunks = per_w // chunk
    n_vec = chunk * d // 16
    mesh = plsc.VectorSubcoreMesh(core_axis_name="c", subcore_axis_name="s")

    @functools.partial(
        pl.kernel, mesh=mesh,
        out_type=jax.ShapeDtypeStruct((n_tok, d), jnp.float32),
        scratch_types=[
            pltpu.VMEM((per_w,), jnp.int32),
            pltpu.VMEM((per_w,), jnp.int32),
        ] + [pltpu.VMEM((chunk, d), jnp.float32) for _ in range(2 * nbuf)]
        + [pltpu.VMEM((chunk, 16), jnp.float32) for _ in range(2 * nbuf)]
        + [pltpu.SemaphoreType.DMA for _ in range(3 * nbuf)],
    )
    def k(rows_hbm, pa_hbm, pb_hbm, wa_hbm, wb_hbm, out_hbm,
          ia_v, ib_v, *rest):
        ba = rest[:nbuf]
        bb = rest[nbuf:2 * nbuf]
        wca = rest[2 * nbuf:3 * nbuf]
        wcb = rest[3 * nbuf:4 * nbuf]
        sa = rest[4 * nbuf:5 * nbuf]
        sb = rest[5 * nbuf:6 * nbuf]
        so = rest[6 * nbuf:]
        wid = lax.axis_index("s") * info.num_cores + lax.axis_index("c")
        base = wid * per_w
        pltpu.sync_copy(pa_hbm.at[pl.ds(base, per_w)], ia_v)
        pltpu.sync_copy(pb_hbm.at[pl.ds(base, per_w)], ib_v)

        def start(c):
            b = c % nbuf
            return (
                pltpu.async_copy(
                    rows_hbm.at[ia_v.at[pl.ds(c * chunk, chunk)]],
                    ba[b], sa[b]),
                pltpu.async_copy(
                    rows_hbm.at[ib_v.at[pl.ds(c * chunk, chunk)]],
                    bb[b], sb[b]),
                pltpu.async_copy(
                    wa_hbm.at[pl.ds(base + c * chunk, chunk)], wca[b], sa[b]),
                pltpu.async_copy(
                    wb_hbm.at[pl.ds(base + c * chunk, chunk)], wcb[b], sb[b]),
            )

        copies = {}
        for c in range(min(nbuf, n_chunks)):
            copies[c] = start(c)
        for c in range(n_chunks):
            b = c % nbuf
            ca, cb, cwa, cwb = copies.pop(c)
            ca.wait()
            cb.wait()
            cwa.wait()
            cwb.wait()

            def row_fn(r, _2, b=b):
                wav = wca[b][r, :]
                wbv = wcb[b][r, :]

                def col_fn(j, _3, r=r, wav=wav, wbv=wbv, b=b):
                    off = j * 16
                    ba[b][r, pl.ds(off, 16)] = (
                        wav * ba[b][r, pl.ds(off, 16)]
                        + wbv * bb[b][r, pl.ds(off, 16)]
                    )
                    return _3

                return lax.fori_loop(0, d // 16, col_fn, _2)

            lax.fori_loop(0, chunk, row_fn, 0)
            pltpu.async_copy(
                ba[b], out_hbm.at[pl.ds(base + c * chunk, chunk)],
                so[b]).wait()
            if c + nbuf < n_chunks:
                copies[c + nbuf] = start(c + nbuf)

    return k


def kernel(x, router_W, router_b, W1, b1, W2, b2):
    Bb, Tt, C = x.shape
    x_flat = x.reshape(_N, _D)
    router_Wp = jnp.pad(router_W, ((0, 0), (0, _EP - _E)))
    router_bp = jnp.pad(router_b, (0, _EP - _E)).reshape(1, _EP)

    ti_full, tv_full, imp = _run_router(x_flat, router_Wp, router_bp)
    ti2 = ti_full[:, :_K]
    tv2 = tv_full[:, :_K]

    e_seq = ti2.reshape(_A)
    oh = (e_seq[:, None] == jnp.arange(_E, dtype=jnp.int32)[None, :]).astype(
        jnp.int32)
    cs = jnp.cumsum(oh, axis=0)
    rank = jnp.take_along_axis(cs - oh, e_seq[:, None], axis=1)[:, 0]
    counts = cs[-1]
    psize = ((counts + _TM - 1) // _TM) * _TM
    pstart = jnp.concatenate(
        [jnp.zeros((1,), jnp.int32), jnp.cumsum(psize)[:-1]])
    dest = pstart[e_seq] + rank

    tok = jnp.arange(_A, dtype=jnp.int32) // _K
    tok_sorted = (jnp.arange(_P, dtype=jnp.int32) % _N).at[dest].set(tok)

    tile_starts = jnp.arange(_NT, dtype=jnp.int32) * _TM
    ends = pstart + psize
    in_range = (tile_starts[:, None] >= pstart[None, :]) & (
        tile_starts[:, None] < ends[None, :])
    expert_tile = jnp.sum(
        jnp.where(in_range, jnp.arange(_E, dtype=jnp.int32)[None, :], 0),
        axis=1)

    x_sorted = _make_sc_gather(_P, _D, 16, nbuf=6)(x_flat, tok_sorted)
    b1r = b1.reshape(_E * _NF, 1, _F)
    b2r = b2.reshape(_E, 1, _D)
    out_sorted = _run_ffn(expert_tile, x_sorted,
                          W1.astype(jnp.bfloat16), b1r,
                          W2.astype(jnp.bfloat16), b2r)
    pos = dest.reshape(_N, _K)
    wa2 = jnp.broadcast_to(tv2[:, 0:1], (_N, 16))
    wb2 = jnp.broadcast_to(tv2[:, 1:2], (_N, 16))
    output = _make_sc_combine(_N, _D, 16)(
        out_sorted, pos[:, 0], pos[:, 1], wa2, wb2)

    importance = imp[0, :_E] / _N
    load = (counts.astype(jnp.float32) / _A) / _A
    aux_loss = _E * jnp.sum(importance * load)
    return output.reshape(Bb, Tt, C), aux_loss


# speedup vs baseline: 1.1327x; 1.1327x over previous
import functools
import jax
import jax.numpy as jnp
from jax import lax
from jax.experimental import pallas as pl
from jax.experimental.pallas import tpu as pltpu
from jax.experimental.pallas import tpu_sc as plsc

_B, _T, _D = 4, 2048, 1024
_E, _K, _DFF = 8, 2, 4096
_N = _B * _T
_A = _N * _K
_TM = 512
_P = _A + _E * _TM
_NT = _P // _TM
_F = 2048
_NF = _DFF // _F
_EP = 128


def _router_body(x_ref, w_ref, b_ref, ti_ref, tv_ref, imp_ref):
    i = pl.program_id(0)
    x = x_ref[...]
    logits = jnp.dot(x, w_ref[...], preferred_element_type=jnp.float32)
    logits = logits + b_ref[...]
    col = lax.broadcasted_iota(jnp.int32, logits.shape, 1)
    logits = jnp.where(col < _E, logits, -1e30)
    m = jnp.max(logits, axis=-1, keepdims=True)
    p = jnp.exp(logits - m)
    probs = p / jnp.sum(p, axis=-1, keepdims=True)

    v0 = jnp.max(probs, axis=-1, keepdims=True)
    e0 = jnp.min(jnp.where(probs == v0, col, _EP), axis=-1, keepdims=True)
    probs2 = jnp.where(col == e0, -1.0, probs)
    v1 = jnp.max(probs2, axis=-1, keepdims=True)
    e1 = jnp.min(jnp.where(probs2 == v1, col, _EP), axis=-1, keepdims=True)
    s = v0 + v1
    w0 = v0 / s
    w1 = v1 / s

    ti_ref[...] = jnp.where(col == 0, e0, jnp.where(col == 1, e1, 0))
    tv_ref[...] = jnp.where(col == 0, w0, jnp.where(col == 1, w1, 0.0))

    @pl.when(i == 0)
    def _():
        imp_ref[...] = jnp.zeros_like(imp_ref)
    imp_ref[...] += jnp.sum(probs, axis=0, keepdims=True)


def _run_router(x_flat, router_Wp, router_bp):
    tb = 1024
    return pl.pallas_call(
        _router_body,
        grid=(_N // tb,),
        in_specs=[
            pl.BlockSpec((tb, _D), lambda i: (i, 0)),
            pl.BlockSpec((_D, _EP), lambda i: (0, 0)),
            pl.BlockSpec((1, _EP), lambda i: (0, 0)),
        ],
        out_specs=[
            pl.BlockSpec((tb, _EP), lambda i: (i, 0)),
            pl.BlockSpec((tb, _EP), lambda i: (i, 0)),
            pl.BlockSpec((1, _EP), lambda i: (0, 0)),
        ],
        out_shape=[
            jax.ShapeDtypeStruct((_N, _EP), jnp.int32),
            jax.ShapeDtypeStruct((_N, _EP), jnp.float32),
            jax.ShapeDtypeStruct((1, _EP), jnp.float32),
        ],
    )(x_flat, router_Wp, router_bp)


def _make_sc_gather(n_rows, d, chunk, nbuf=3):
    info = plsc.get_sparse_core_info()
    nw = info.num_cores * info.num_subcores
    per_w = n_rows // nw
    n_chunks = per_w // chunk
    mesh = plsc.VectorSubcoreMesh(core_axis_name="c", subcore_axis_name="s")

    @functools.partial(
        pl.kernel, mesh=mesh,
        out_type=jax.ShapeDtypeStruct((n_rows, d), jnp.float32),
        scratch_types=[
            pltpu.VMEM((per_w,), jnp.int32),
        ] + [pltpu.VMEM((chunk, d), jnp.float32) for _ in range(nbuf)]
        + [pltpu.SemaphoreType.DMA for _ in range(2 * nbuf)],
    )
    def k(table_hbm, idx_hbm, out_hbm, idx_v, *bufs_sems):
        bufs = bufs_sems[:nbuf]
        gsems = bufs_sems[nbuf:2 * nbuf]
        ssems = bufs_sems[2 * nbuf:]
        wid = lax.axis_index("s") * info.num_cores + lax.axis_index("c")
        base = wid * per_w
        pltpu.sync_copy(idx_hbm.at[pl.ds(base, per_w)], idx_v)

        def start(c):
            b = c % nbuf
            return pltpu.async_copy(
                table_hbm.at[idx_v.at[pl.ds(c * chunk, chunk)]],
                bufs[b], gsems[b])

        copies = {}
        for c in range(min(nbuf, n_chunks)):
            copies[c] = start(c)
        for c in range(n_chunks):
            b = c % nbuf
            copies.pop(c).wait()
            st = pltpu.async_copy(
                bufs[b], out_hbm.at[pl.ds(base + c * chunk, chunk)], ssems[b])
            if c + nbuf < n_chunks:
                st.wait()
                copies[c + nbuf] = start(c + nbuf)
            else:
                st.wait()

    return k


def _ffn_body(eid_ref, x_ref, w1_ref, b1_ref, w2_ref, b2_ref,
              out_ref, acc_ref):
    f = pl.program_id(1)
    xb = x_ref[...].astype(jnp.bfloat16)
    h = jnp.dot(xb, w1_ref[0].astype(jnp.bfloat16),
                preferred_element_type=jnp.float32)
    h = jnp.maximum(h + b1_ref[0], 0.0)
    contrib = jnp.dot(h.astype(jnp.bfloat16), w2_ref[0].astype(jnp.bfloat16),
                      preferred_element_type=jnp.float32)

    @pl.when(f == 0)
    def _():
        acc_ref[...] = jnp.zeros_like(acc_ref)
    acc_ref[...] += contrib

    @pl.when(f == _NF - 1)
    def _():
        out_ref[...] = acc_ref[...] + b2_ref[0]


def _run_ffn(expert_tile, x_sorted, W1, b1, W2, b2):
    grid_spec = pltpu.PrefetchScalarGridSpec(
        num_scalar_prefetch=1,
        grid=(_NT, _NF),
        in_specs=[
            pl.BlockSpec((_TM, _D), lambda i, f, e: (i, 0)),
            pl.BlockSpec((1, _D, _F), lambda i, f, e: (e[i], 0, f)),
            pl.BlockSpec((1, 1, _F), lambda i, f, e: (e[i] * _NF + f, 0, 0)),
            pl.BlockSpec((1, _F, _D), lambda i, f, e: (e[i], f, 0)),
            pl.BlockSpec((1, 1, _D), lambda i, f, e: (e[i], 0, 0)),
        ],
        out_specs=pl.BlockSpec((_TM, _D), lambda i, f, e: (i, 0)),
        scratch_shapes=[pltpu.VMEM((_TM, _D), jnp.float32)],
    )
    return pl.pallas_call(
        _ffn_body,
        grid_spec=grid_spec,
        out_shape=jax.ShapeDtypeStruct((_P, _D), jnp.float32),
        compiler_params=pltpu.CompilerParams(
            dimension_semantics=("arbitrary", "arbitrary"),
        ),
    )(expert_tile, x_sorted, W1, b1, W2, b2)


def _make_sc_combine(n_tok, d, chunk, nbuf=2):
    info = plsc.get_sparse_core_info()
    nw = info.num_cores * info.num_subcores
    per_w = n_tok // nw
    n_chunks = per_w // chunk
    n_vec = chunk * d // 16
    mesh = plsc.VectorSubcoreMesh(core_axis_name="c", subcore_axis_name="s")

    @functools.partial(
        pl.kernel, mesh=mesh,
        out_type=jax.ShapeDtypeStruct((n_tok, d), jnp.float32),
        scratch_types=[
            pltpu.VMEM((per_w,), jnp.int32),
            pltpu.VMEM((per_w,), jnp.int32),
        ] + [pltpu.VMEM((chunk, d), jnp.float32) for _ in range(2 * nbuf)]
        + [pltpu.VMEM((chunk, 16), jnp.float32) for _ in range(2 * nbuf)]
        + [pltpu.SemaphoreType.DMA for _ in range(3 * nbuf)],
    )
    def k(rows_hbm, pa_hbm, pb_hbm, wa_hbm, wb_hbm, out_hbm,
          ia_v, ib_v, *rest):
        ba = rest[:nbuf]
        bb = rest[nbuf:2 * nbuf]
        wca = rest[2 * nbuf:3 * nbuf]
        wcb = rest[3 * nbuf:4 * nbuf]
        sa = rest[4 * nbuf:5 * nbuf]
        sb = rest[5 * nbuf:6 * nbuf]
        so = rest[6 * nbuf:]
        wid = lax.axis_index("s") * info.num_cores + lax.axis_index("c")
        base = wid * per_w
        pltpu.sync_copy(pa_hbm.at[pl.ds(base, per_w)], ia_v)
        pltpu.sync_copy(pb_hbm.at[pl.ds(base, per_w)], ib_v)

        def start(c):
            b = c % nbuf
            return (
                pltpu.async_copy(
                    rows_hbm.at[ia_v.at[pl.ds(c * chunk, chunk)]],
                    ba[b], sa[b]),
                pltpu.async_copy(
                    rows_hbm.at[ib_v.at[pl.ds(c * chunk, chunk)]],
                    bb[b], sb[b]),
                pltpu.async_copy(
                    wa_hbm.at[pl.ds(base + c * chunk, chunk)], wca[b], sa[b]),
                pltpu.async_copy(
                    wb_hbm.at[pl.ds(base + c * chunk, chunk)], wcb[b], sb[b]),
            )

        copies = {}
        for c in range(min(nbuf, n_chunks)):
            copies[c] = start(c)
        for c in range(n_chunks):
            b = c % nbuf
            ca, cb, cwa, cwb = copies.pop(c)
            ca.wait()
            cb.wait()
            cwa.wait()
            cwb.wait()

            def row_fn(r, _2, b=b):
                wav = wca[b][r, :]
                wbv = wcb[b][r, :]

                def col_fn(j, _3, r=r, wav=wav, wbv=wbv, b=b):
                    off = j * 16
                    ba[b][r, pl.ds(off, 16)] = (
                        wav * ba[b][r, pl.ds(off, 16)]
                        + wbv * bb[b][r, pl.ds(off, 16)]
                    )
                    return _3

                return lax.fori_loop(0, d // 16, col_fn, _2)

            lax.fori_loop(0, chunk, row_fn, 0)
            pltpu.async_copy(
                ba[b], out_hbm.at[pl.ds(base + c * chunk, chunk)],
                so[b]).wait()
            if c + nbuf < n_chunks:
                copies[c + nbuf] = start(c + nbuf)

    return k


def kernel(x, router_W, router_b, W1, b1, W2, b2):
    Bb, Tt, C = x.shape
    x_flat = x.reshape(_N, _D)
    router_Wp = jnp.pad(router_W, ((0, 0), (0, _EP - _E)))
    router_bp = jnp.pad(router_b, (0, _EP - _E)).reshape(1, _EP)

    ti_full, tv_full, imp = _run_router(x_flat, router_Wp, router_bp)
    ti2 = ti_full[:, :_K]
    tv2 = tv_full[:, :_K]

    e_seq = ti2.reshape(_A)
    oh = (e_seq[:, None] == jnp.arange(_E, dtype=jnp.int32)[None, :]).astype(
        jnp.int32)
    cs = jnp.cumsum(oh, axis=0)
    rank = jnp.take_along_axis(cs - oh, e_seq[:, None], axis=1)[:, 0]
    counts = cs[-1]
    psize = ((counts + _TM - 1) // _TM) * _TM
    pstart = jnp.concatenate(
        [jnp.zeros((1,), jnp.int32), jnp.cumsum(psize)[:-1]])
    dest = pstart[e_seq] + rank

    tok = jnp.arange(_A, dtype=jnp.int32) // _K
    tok_sorted = (jnp.arange(_P, dtype=jnp.int32) % _N).at[dest].set(tok)

    tile_starts = jnp.arange(_NT, dtype=jnp.int32) * _TM
    ends = pstart + psize
    in_range = (tile_starts[:, None] >= pstart[None, :]) & (
        tile_starts[:, None] < ends[None, :])
    expert_tile = jnp.sum(
        jnp.where(in_range, jnp.arange(_E, dtype=jnp.int32)[None, :], 0),
        axis=1)

    x_sorted = _make_sc_gather(_P, _D, 16, nbuf=6)(x_flat, tok_sorted)
    b1r = b1.reshape(_E * _NF, 1, _F)
    b2r = b2.reshape(_E, 1, _D)
    out_sorted = _run_ffn(expert_tile, x_sorted, W1, b1r, W2, b2r)
    pos = dest.reshape(_N, _K)
    wa2 = jnp.broadcast_to(tv2[:, 0:1], (_N, 16))
    wb2 = jnp.broadcast_to(tv2[:, 1:2], (_N, 16))
    output = _make_sc_combine(_N, _D, 16)(
        out_sorted, pos[:, 0], pos[:, 1], wa2, wb2)

    importance = imp[0, :_E] / _N
    load = (counts.astype(jnp.float32) / _A) / _A
    aux_loss = _E * jnp.sum(importance * load)
    return output.reshape(Bb, Tt, C), aux_loss
